# baseline (speedup 1.0000x reference)
import functools

import jax
import jax.numpy as jnp
from jax.experimental import pallas as pl
from jax.experimental.pallas import tpu as pltpu

_K = 3
_J = 3
_NU = 0.001
_ALPHA = 0.1
_NN = _K + _J

_NET16 = [
    (0, 1), (2, 3), (4, 5), (8, 9), (10, 11), (12, 13), (14, 15),
    (0, 2), (1, 3), (4, 6), (5, 7), (8, 10), (9, 11), (12, 14), (13, 15),
    (1, 2), (9, 10), (13, 14),
    (0, 4), (1, 5), (2, 6), (3, 7), (8, 12), (9, 13), (10, 14), (11, 15),
    (2, 4), (3, 5), (10, 12), (11, 13),
    (1, 2), (3, 4), (13, 14),
    (0, 8), (1, 9), (2, 10), (3, 11), (4, 12), (5, 13),
    (4, 8), (5, 9), (6, 10),
    (3, 5), (6, 8), (5, 6),
]

_NET8 = [
    (0, 1), (2, 3), (4, 5), (0, 2), (1, 3), (4, 6), (5, 7),
    (1, 2), (2, 6), (3, 7), (3, 5), (5, 6),
]


def _apply_net(arrs, net):
    arrs = list(arrs)
    for i, j in net:
        a, b = arrs[i], arrs[j]
        arrs[i] = jnp.minimum(a, b)
        arrs[j] = jnp.maximum(a, b)
    return arrs[:_NN]


def _tournament(chunks):
    while len(chunks) >= 16:
        kept = []
        g = len(chunks) // 16
        for gi in range(g):
            kept.extend(_apply_net(chunks[gi * 16:(gi + 1) * 16], _NET16))
        kept.extend(chunks[g * 16:])
        chunks = kept
    if len(chunks) == 8:
        chunks = _apply_net(chunks, _NET8)
    return chunks


def _cfa_block_kernel(x_ref, c_ref, r_ref, out_ref, caug_ref):
    x = x_ref[...]
    r2 = r_ref[0, 0] * r_ref[0, 0]
    rows, d = x.shape
    daug = caug_ref.shape[0]

    @pl.when(pl.program_id(0) == 0)
    def _norms():
        c = c_ref[...]
        caug_ref[0:d, :] = c.astype(jnp.bfloat16)
        caug_ref[d:d + 1, :] = jnp.sum(
            c * c, axis=0, keepdims=True).astype(jnp.bfloat16)
        caug_ref[d + 1:, :] = jnp.zeros((daug - d - 1, c.shape[1]),
                                        jnp.bfloat16)

    x_aug = jnp.concatenate(
        [x * -2.0,
         jnp.ones((rows, 1), jnp.float32),
         jnp.zeros((rows, daug - d - 1), jnp.float32)],
        axis=1).astype(jnp.bfloat16)

    dist = jnp.dot(x_aug, caug_ref[...],
                   preferred_element_type=jnp.float32).astype(jnp.bfloat16)
    x2 = jnp.sum(x * x, axis=1, keepdims=True)

    rows, width = dist.shape
    if width % 128 == 0 and width // 128 >= 2:
        chunks = [dist[:, i * 128:(i + 1) * 128] for i in range(width // 128)]
        chunks = _tournament(chunks)
        work = jnp.concatenate(chunks, axis=1)
    else:
        work = dist

    filled = jnp.zeros((rows, 1), dtype=jnp.float32)
    acc = jnp.zeros((rows, 1), dtype=jnp.float32)
    for _ in range(_NN):
        m = jnp.min(work, axis=1, keepdims=True)
        eq = work == m
        cnt = jnp.sum(eq.astype(jnp.bfloat16), axis=1,
                      keepdims=True).astype(jnp.float32)
        work = jnp.where(eq, jnp.inf, work)
        take = jnp.minimum(cnt, _NN - filled)
        n_att = jnp.minimum(filled + take, float(_K)) - jnp.minimum(filled, float(_K))
        n_rep = take - n_att
        d_k = jnp.where(take > 0.0, m.astype(jnp.float32) + x2, 0.0)
        acc = acc + n_att * jnp.maximum(d_k - r2, 0.0)
        acc = acc + n_rep * jnp.maximum(r2 - d_k - _ALPHA, 0.0)
        filled = filled + take

    partial = jnp.sum(acc).reshape(1, 1)

    @pl.when(pl.program_id(0) == 0)
    def _init():
        out_ref[...] = partial

    @pl.when(pl.program_id(0) != 0)
    def _accum():
        out_ref[...] += partial


@functools.partial(jax.jit, static_argnames=())
def _cfa_loss(x, c, r):
    rows, d = x.shape
    m = c.shape[1]
    block_r = 128
    assert rows % block_r == 0
    grid = (rows // block_r,)
    r11 = r.reshape(1, 1)

    total = pl.pallas_call(
        _cfa_block_kernel,
        grid=grid,
        in_specs=[
            pl.BlockSpec((block_r, d), lambda i: (i, 0)),
            pl.BlockSpec((d, m), lambda i: (0, 0)),
            pl.BlockSpec((1, 1), lambda i: (0, 0)),
        ],
        out_specs=pl.BlockSpec((1, 1), lambda i: (0, 0)),
        out_shape=jax.ShapeDtypeStruct((1, 1), jnp.float32),
        scratch_shapes=[pltpu.VMEM((((d + 1 + 7) // 8) * 8, m), jnp.bfloat16)],
    )(x, c, r11)

    denom = float(rows * _K)
    return (1.0 / _NU) * total[0, 0] / denom


def kernel(phi_p, C, r):
    b, hw, d = phi_p.shape
    x = phi_p.reshape(b * hw, d)
    return _cfa_loss(x, C, r)


# speedup vs baseline: 1.0893x; 1.0893x over previous
import functools

import jax
import jax.numpy as jnp
from jax.experimental import pallas as pl
from jax.experimental.pallas import tpu as pltpu

_K = 3
_J = 3
_NU = 0.001
_ALPHA = 0.1
_NN = _K + _J

_NET16 = [
    (0, 1), (2, 3), (4, 5), (8, 9), (10, 11), (12, 13), (14, 15),
    (0, 2), (1, 3), (4, 6), (5, 7), (8, 10), (9, 11), (12, 14), (13, 15),
    (1, 2), (9, 10), (13, 14),
    (0, 4), (1, 5), (2, 6), (3, 7), (8, 12), (9, 13), (10, 14), (11, 15),
    (2, 4), (3, 5), (10, 12), (11, 13),
    (1, 2), (3, 4), (13, 14),
    (0, 8), (1, 9), (2, 10), (3, 11), (4, 12), (5, 13),
    (4, 8), (5, 9), (6, 10),
    (3, 5), (6, 8), (5, 6),
]

_NET8 = [
    (0, 1), (2, 3), (4, 5), (0, 2), (1, 3), (4, 6), (5, 7),
    (1, 2), (2, 6), (3, 7), (3, 5), (5, 6),
]


def _apply_net(arrs, net):
    arrs = list(arrs)
    for i, j in net:
        a, b = arrs[i], arrs[j]
        arrs[i] = jnp.minimum(a, b)
        arrs[j] = jnp.maximum(a, b)
    return arrs[:_NN]


def _tournament(chunks):
    while len(chunks) >= 16:
        kept = []
        g = len(chunks) // 16
        for gi in range(g):
            kept.extend(_apply_net(chunks[gi * 16:(gi + 1) * 16], _NET16))
        kept.extend(chunks[g * 16:])
        chunks = kept
    if len(chunks) == 8:
        chunks = _apply_net(chunks, _NET8)
    return chunks


def _cfa_block_kernel(x_ref, c_ref, r_ref, out_ref, caug_ref):
    x = x_ref[...]
    r2 = r_ref[0, 0] * r_ref[0, 0]
    rows, d = x.shape
    daug = caug_ref.shape[0]

    @pl.when(pl.program_id(0) == 0)
    def _norms():
        c = c_ref[...]
        caug_ref[0:d, :] = c
        caug_ref[d:d + 1, :] = jnp.sum(c * c, axis=0, keepdims=True)
        caug_ref[d + 1:, :] = jnp.zeros((daug - d - 1, c.shape[1]), jnp.float32)

    x_aug = jnp.concatenate(
        [x * -2.0,
         jnp.ones((rows, 1), jnp.float32),
         jnp.zeros((rows, daug - d - 1), jnp.float32)], axis=1)

    dist = jnp.dot(x_aug, caug_ref[...],
                   preferred_element_type=jnp.float32).astype(jnp.bfloat16)
    x2 = jnp.sum(x * x, axis=1, keepdims=True)

    rows, width = dist.shape
    if width % 128 == 0 and width // 128 >= 2:
        chunks = [dist[:, i * 128:(i + 1) * 128] for i in range(width // 128)]
        chunks = _tournament(chunks)
        work = jnp.concatenate(chunks, axis=1)
    else:
        work = dist

    filled = jnp.zeros((rows, 1), dtype=jnp.float32)
    acc = jnp.zeros((rows, 1), dtype=jnp.float32)
    for _ in range(_NN):
        m = jnp.min(work, axis=1, keepdims=True)
        eq = work == m
        cnt = jnp.sum(eq.astype(jnp.bfloat16), axis=1,
                      keepdims=True).astype(jnp.float32)
        work = jnp.where(eq, jnp.inf, work)
        take = jnp.minimum(cnt, _NN - filled)
        n_att = jnp.minimum(filled + take, float(_K)) - jnp.minimum(filled, float(_K))
        n_rep = take - n_att
        d_k = jnp.where(take > 0.0, m.astype(jnp.float32) + x2, 0.0)
        acc = acc + n_att * jnp.maximum(d_k - r2, 0.0)
        acc = acc + n_rep * jnp.maximum(r2 - d_k - _ALPHA, 0.0)
        filled = filled + take

    partial = jnp.sum(acc).reshape(1, 1)

    @pl.when(pl.program_id(0) == 0)
    def _init():
        out_ref[...] = partial

    @pl.when(pl.program_id(0) != 0)
    def _accum():
        out_ref[...] += partial


@functools.partial(jax.jit, static_argnames=())
def _cfa_loss(x, c, r):
    rows, d = x.shape
    m = c.shape[1]
    block_r = next((b for b in (224, 128) if rows % b == 0), rows)
    assert rows % block_r == 0
    grid = (rows // block_r,)
    r11 = r.reshape(1, 1)

    total = pl.pallas_call(
        _cfa_block_kernel,
        grid=grid,
        in_specs=[
            pl.BlockSpec((block_r, d), lambda i: (i, 0)),
            pl.BlockSpec((d, m), lambda i: (0, 0)),
            pl.BlockSpec((1, 1), lambda i: (0, 0)),
        ],
        out_specs=pl.BlockSpec((1, 1), lambda i: (0, 0)),
        out_shape=jax.ShapeDtypeStruct((1, 1), jnp.float32),
        scratch_shapes=[pltpu.VMEM((((d + 1 + 7) // 8) * 8, m), jnp.float32)],
    )(x, c, r11)

    denom = float(rows * _K)
    return (1.0 / _NU) * total[0, 0] / denom


def kernel(phi_p, C, r):
    b, hw, d = phi_p.shape
    x = phi_p.reshape(b * hw, d)
    return _cfa_loss(x, C, r)


# speedup vs baseline: 1.1348x; 1.0418x over previous
import functools

import jax
import jax.numpy as jnp
from jax.experimental import pallas as pl
from jax.experimental.pallas import tpu as pltpu

_K = 3
_J = 3
_NU = 0.001
_ALPHA = 0.1
_NN = _K + _J

_NET16 = [
    (0, 1), (2, 3), (4, 5), (8, 9), (10, 11), (12, 13), (14, 15),
    (0, 2), (1, 3), (4, 6), (5, 7), (8, 10), (9, 11), (12, 14), (13, 15),
    (1, 2), (9, 10), (13, 14),
    (0, 4), (1, 5), (2, 6), (3, 7), (8, 12), (9, 13), (10, 14), (11, 15),
    (2, 4), (3, 5), (10, 12), (11, 13),
    (1, 2), (3, 4), (13, 14),
    (0, 8), (1, 9), (2, 10), (3, 11), (4, 12), (5, 13),
    (4, 8), (5, 9), (6, 10),
    (3, 5), (6, 8), (5, 6),
]

_NET8 = [
    (0, 1), (2, 3), (4, 5), (0, 2), (1, 3), (4, 6), (5, 7),
    (1, 2), (2, 6), (3, 7), (3, 5), (5, 6),
]


def _apply_net(arrs, net):
    arrs = list(arrs)
    for i, j in net:
        a, b = arrs[i], arrs[j]
        arrs[i] = jnp.minimum(a, b)
        arrs[j] = jnp.maximum(a, b)
    return arrs[:_NN]


def _tournament(chunks):
    while len(chunks) >= 16:
        kept = []
        g = len(chunks) // 16
        for gi in range(g):
            kept.extend(_apply_net(chunks[gi * 16:(gi + 1) * 16], _NET16))
        kept.extend(chunks[g * 16:])
        chunks = kept
    if len(chunks) == 8:
        chunks = _apply_net(chunks, _NET8)
    return chunks


def _cfa_block_kernel(x_ref, c_ref, r_ref, out_ref, caug_ref):
    x = x_ref[...]
    r2 = r_ref[0, 0] * r_ref[0, 0]
    rows, d = x.shape
    daug = caug_ref.shape[0]

    @pl.when(pl.program_id(0) == 0)
    def _norms():
        c = c_ref[...]
        caug_ref[0:d, :] = c
        caug_ref[d:d + 1, :] = jnp.sum(c * c, axis=0, keepdims=True)
        caug_ref[d + 1:, :] = jnp.zeros((daug - d - 1, c.shape[1]), jnp.float32)

    x_aug = jnp.concatenate(
        [x * -2.0,
         jnp.ones((rows, 1), jnp.float32),
         jnp.zeros((rows, daug - d - 1), jnp.float32)], axis=1)

    dist = jnp.dot(x_aug, caug_ref[...],
                   preferred_element_type=jnp.float32).astype(jnp.bfloat16)
    x2 = jnp.sum(x * x, axis=1, keepdims=True)

    rows, width = dist.shape
    if width % 128 == 0 and width // 128 >= 2:
        chunks = [dist[:, i * 128:(i + 1) * 128] for i in range(width // 128)]
        chunks = _tournament(chunks)
        work = jnp.concatenate(chunks, axis=1)
    else:
        work = dist

    filled = jnp.zeros((rows, 1), dtype=jnp.float32)
    acc = jnp.zeros((rows, 1), dtype=jnp.float32)
    for _ in range(_NN):
        m = jnp.min(work, axis=1, keepdims=True)
        eq = work == m
        cnt = jnp.sum(eq.astype(jnp.bfloat16), axis=1,
                      keepdims=True).astype(jnp.float32)
        work = jnp.where(eq, jnp.inf, work)
        take = jnp.minimum(cnt, _NN - filled)
        n_att = jnp.minimum(filled + take, float(_K)) - jnp.minimum(filled, float(_K))
        n_rep = take - n_att
        d_k = jnp.where(take > 0.0, m.astype(jnp.float32) + x2, 0.0)
        acc = acc + n_att * jnp.maximum(d_k - r2, 0.0)
        acc = acc + n_rep * jnp.maximum(r2 - d_k - _ALPHA, 0.0)
        filled = filled + take

    partial = jnp.sum(acc).reshape(1, 1)

    @pl.when(pl.program_id(0) == 0)
    def _init():
        out_ref[...] = partial

    @pl.when(pl.program_id(0) != 0)
    def _accum():
        out_ref[...] += partial


@functools.partial(jax.jit, static_argnames=())
def _cfa_loss(x, c, r):
    rows, d = x.shape
    m = c.shape[1]
    block_r = next((b for b in (448, 224, 128) if rows % b == 0), rows)
    assert rows % block_r == 0
    grid = (rows // block_r,)
    r11 = r.reshape(1, 1)

    total = pl.pallas_call(
        _cfa_block_kernel,
        grid=grid,
        in_specs=[
            pl.BlockSpec((block_r, d), lambda i: (i, 0)),
            pl.BlockSpec((d, m), lambda i: (0, 0)),
            pl.BlockSpec((1, 1), lambda i: (0, 0)),
        ],
        out_specs=pl.BlockSpec((1, 1), lambda i: (0, 0)),
        out_shape=jax.ShapeDtypeStruct((1, 1), jnp.float32),
        scratch_shapes=[pltpu.VMEM((((d + 1 + 7) // 8) * 8, m), jnp.float32)],
    )(x, c, r11)

    denom = float(rows * _K)
    return (1.0 / _NU) * total[0, 0] / denom


def kernel(phi_p, C, r):
    b, hw, d = phi_p.shape
    x = phi_p.reshape(b * hw, d)
    return _cfa_loss(x, C, r)


# speedup vs baseline: 1.1502x; 1.0135x over previous
import functools

import jax
import jax.numpy as jnp
from jax.experimental import pallas as pl
from jax.experimental.pallas import tpu as pltpu

_K = 3
_J = 3
_NU = 0.001
_ALPHA = 0.1
_NN = _K + _J

_NET16 = [
    (0, 1), (2, 3), (4, 5), (8, 9), (10, 11), (12, 13), (14, 15),
    (0, 2), (1, 3), (4, 6), (5, 7), (8, 10), (9, 11), (12, 14), (13, 15),
    (1, 2), (9, 10), (13, 14),
    (0, 4), (1, 5), (2, 6), (3, 7), (8, 12), (9, 13), (10, 14), (11, 15),
    (2, 4), (3, 5), (10, 12), (11, 13),
    (1, 2), (3, 4), (13, 14),
    (0, 8), (1, 9), (2, 10), (3, 11), (4, 12), (5, 13),
    (4, 8), (5, 9), (6, 10),
    (3, 5), (6, 8), (5, 6),
]

_NET8 = [
    (0, 1), (2, 3), (4, 5), (0, 2), (1, 3), (4, 6), (5, 7),
    (1, 2), (2, 6), (3, 7), (3, 5), (5, 6),
]


def _apply_net(arrs, net):
    arrs = list(arrs)
    for i, j in net:
        a, b = arrs[i], arrs[j]
        arrs[i] = jnp.minimum(a, b)
        arrs[j] = jnp.maximum(a, b)
    return arrs[:_NN]


def _tournament(chunks):
    while len(chunks) >= 16:
        kept = []
        g = len(chunks) // 16
        for gi in range(g):
            kept.extend(_apply_net(chunks[gi * 16:(gi + 1) * 16], _NET16))
        kept.extend(chunks[g * 16:])
        chunks = kept
    if len(chunks) == 8:
        chunks = _apply_net(chunks, _NET8)
    return chunks


def _cfa_block_kernel(x_ref, c_ref, r_ref, out_ref, caug_ref):
    x = x_ref[...]
    r2 = r_ref[0, 0] * r_ref[0, 0]
    rows, d = x.shape
    daug = caug_ref.shape[0]

    @pl.when(pl.program_id(0) == 0)
    def _norms():
        c = c_ref[...]
        caug_ref[0:d, :] = c
        caug_ref[d:d + 1, :] = jnp.sum(c * c, axis=0, keepdims=True)
        caug_ref[d + 1:, :] = jnp.zeros((daug - d - 1, c.shape[1]), jnp.float32)

    x_aug = jnp.concatenate(
        [x * -2.0,
         jnp.ones((rows, 1), jnp.float32),
         jnp.zeros((rows, daug - d - 1), jnp.float32)], axis=1)

    dist = jnp.dot(x_aug, caug_ref[...],
                   preferred_element_type=jnp.float32).astype(jnp.bfloat16)
    x2 = jnp.sum(x * x, axis=1, keepdims=True)

    rows, width = dist.shape
    if width % 128 == 0 and width // 128 >= 2:
        chunks = [dist[:, i * 128:(i + 1) * 128] for i in range(width // 128)]
        chunks = _tournament(chunks)
        work = jnp.concatenate(chunks, axis=1)
    else:
        work = dist

    filled = jnp.zeros((rows, 1), dtype=jnp.float32)
    acc = jnp.zeros((rows, 1), dtype=jnp.float32)
    for _ in range(_NN):
        m = jnp.min(work, axis=1, keepdims=True)
        eq = work == m
        cnt = jnp.sum(eq.astype(jnp.bfloat16), axis=1,
                      keepdims=True).astype(jnp.float32)
        work = jnp.where(eq, jnp.inf, work)
        take = jnp.minimum(cnt, _NN - filled)
        n_att = jnp.minimum(filled + take, float(_K)) - jnp.minimum(filled, float(_K))
        n_rep = take - n_att
        d_k = jnp.where(take > 0.0, m.astype(jnp.float32) + x2, 0.0)
        acc = acc + n_att * jnp.maximum(d_k - r2, 0.0)
        acc = acc + n_rep * jnp.maximum(r2 - d_k - _ALPHA, 0.0)
        filled = filled + take

    partial = jnp.sum(acc).reshape(1, 1)

    @pl.when(pl.program_id(0) == 0)
    def _init():
        out_ref[...] = partial

    @pl.when(pl.program_id(0) != 0)
    def _accum():
        out_ref[...] += partial


@functools.partial(jax.jit, static_argnames=())
def _cfa_loss(x, c, r):
    rows, d = x.shape
    m = c.shape[1]
    block_r = next((b for b in (896, 448, 224, 128) if rows % b == 0), rows)
    assert rows % block_r == 0
    grid = (rows // block_r,)
    r11 = r.reshape(1, 1)

    total = pl.pallas_call(
        _cfa_block_kernel,
        grid=grid,
        in_specs=[
            pl.BlockSpec((block_r, d), lambda i: (i, 0)),
            pl.BlockSpec((d, m), lambda i: (0, 0)),
            pl.BlockSpec((1, 1), lambda i: (0, 0)),
        ],
        out_specs=pl.BlockSpec((1, 1), lambda i: (0, 0)),
        out_shape=jax.ShapeDtypeStruct((1, 1), jnp.float32),
        scratch_shapes=[pltpu.VMEM((((d + 1 + 7) // 8) * 8, m), jnp.float32)],
    )(x, c, r11)

    denom = float(rows * _K)
    return (1.0 / _NU) * total[0, 0] / denom


def kernel(phi_p, C, r):
    b, hw, d = phi_p.shape
    x = phi_p.reshape(b * hw, d)
    return _cfa_loss(x, C, r)
